# baseline (speedup 1.0000x reference)
import functools

import jax
import jax.numpy as jnp
from jax import lax
from jax.experimental import pallas as pl
from jax.experimental.pallas import tpu as pltpu
from jax.experimental.pallas import tpu_sc as plsc

N_NODES = 10000
D = 128
NC, NS = 2, 16
NW = NC * NS
N_PAD = 10240
TRASH = 10000
E = 320000
CH = 128
EROWS = 2560
CPT = EROWS // NW
E_PAD = EROWS * CH
RPT = N_PAD // NS
DEG_ROWS = N_PAD // 128
DEG_RPT = 8

_MESH = plsc.VectorSubcoreMesh(core_axis_name="c", subcore_axis_name="s")


E_PT = E_PAD // NW


def _deg_body(srcv, dstv, iota_hbm, zeros_hbm, out_o, out_i,
              src_v, dst_v, ho, hi, idx_v, tmp_v, sho, shi, sem1, sem2):
    c = lax.axis_index("c")
    s = lax.axis_index("s")
    wid = s * NC + c
    pltpu.sync_copy(zeros_hbm, ho)
    pltpu.sync_copy(zeros_hbm, hi)
    pltpu.sync_copy(zeros_hbm.at[pl.ds(0, DEG_RPT)], tmp_v)

    @pl.when(s < DEG_ROWS // DEG_RPT)
    def _zero_shared():
        pltpu.sync_copy(tmp_v, sho.at[pl.ds(s * DEG_RPT, DEG_RPT)])
        pltpu.sync_copy(tmp_v, shi.at[pl.ds(s * DEG_RPT, DEG_RPT)])

    pltpu.sync_copy(srcv.at[pl.ds(wid * E_PT, E_PT)], src_v)
    pltpu.sync_copy(dstv.at[pl.ds(wid * E_PT, E_PT)], dst_v)
    pltpu.sync_copy(iota_hbm, idx_v)
    ones = jnp.ones((16,), jnp.float32)

    def body(k, carry):
        s16 = src_v[pl.ds(k * 16, 16)]
        d16 = dst_v[pl.ds(k * 16, 16)]
        plsc.addupdate_scatter(ho, [s16 >> 7, s16 & 127], ones)
        plsc.addupdate_scatter(hi, [d16 >> 7, d16 & 127], ones)
        return carry

    lax.fori_loop(0, E_PT // 16, body, 0)
    plsc.subcore_barrier()
    pltpu.async_copy(ho, sho.at[idx_v.at[0]], sem1, add=True).wait()
    pltpu.async_copy(hi, shi.at[idx_v.at[0]], sem2, add=True).wait()
    plsc.subcore_barrier()

    @pl.when(s < DEG_ROWS // DEG_RPT)
    def _writeback():
        base = c * DEG_ROWS + s * DEG_RPT
        pltpu.sync_copy(sho.at[pl.ds(s * DEG_RPT, DEG_RPT)], tmp_v)
        pltpu.sync_copy(tmp_v, out_o.at[pl.ds(base, DEG_RPT)])
        pltpu.sync_copy(shi.at[pl.ds(s * DEG_RPT, DEG_RPT)], tmp_v)
        pltpu.sync_copy(tmp_v, out_i.at[pl.ds(base, DEG_RPT)])


_deg_kernel = functools.partial(
    pl.kernel,
    out_type=(jax.ShapeDtypeStruct((NC * DEG_ROWS, 128), jnp.float32),
              jax.ShapeDtypeStruct((NC * DEG_ROWS, 128), jnp.float32)),
    mesh=_MESH,
    scratch_types=[
        pltpu.VMEM((E_PT,), jnp.int32),
        pltpu.VMEM((E_PT,), jnp.int32),
        pltpu.VMEM((DEG_ROWS, 128), jnp.float32),
        pltpu.VMEM((DEG_ROWS, 128), jnp.float32),
        pltpu.VMEM((1, DEG_ROWS), jnp.int32),
        pltpu.VMEM((DEG_RPT, 128), jnp.float32),
        pltpu.VMEM_SHARED((DEG_ROWS, 128), jnp.float32),
        pltpu.VMEM_SHARED((DEG_ROWS, 128), jnp.float32),
        pltpu.SemaphoreType.DMA,
        pltpu.SemaphoreType.DMA,
    ],
    compiler_params=pltpu.CompilerParams(needs_layout_passes=False),
)(_deg_body)


CH2 = 128
CROWS_PT = E_PAD // CH2 // NW
BLK = 16


def _edge_body(h_hbm, srcm, dstm, zeros_hbm, out_hbm,
               src_v, dst_v, rows0_v, rows1_v, agg_sh,
               gsem0, gsem1, ssem0, ssem1):
    c = lax.axis_index("c")
    s = lax.axis_index("s")
    wid = s * NC + c
    rows = (rows0_v, rows1_v)
    gsems = (gsem0, gsem1)
    ssems = (ssem0, ssem1)
    for k in range(RPT // CH2):
        pltpu.sync_copy(zeros_hbm, agg_sh.at[pl.ds(s * RPT + k * CH2, CH2)])
    plsc.subcore_barrier()
    for p in range(CROWS_PT // BLK):
        base_row = wid * CROWS_PT + p * BLK
        pltpu.sync_copy(srcm.at[pl.ds(base_row, BLK)], src_v)
        pltpu.sync_copy(dstm.at[pl.ds(base_row, BLK)], dst_v)
        g = pltpu.async_copy(h_hbm.at[src_v.at[0]], rows[0], gsems[0])
        sd = [None, None]
        for j in range(BLK):
            b = j & 1
            nb = b ^ 1
            g.wait()
            if j + 1 < BLK:
                if sd[nb] is not None:
                    sd[nb].wait()
                g = pltpu.async_copy(h_hbm.at[src_v.at[j + 1]], rows[nb],
                                     gsems[nb])
            sd[b] = pltpu.async_copy(rows[b], agg_sh.at[dst_v.at[j]],
                                     ssems[b], add=True)
        sd[0].wait()
        sd[1].wait()
    plsc.subcore_barrier()
    base = c * N_PAD + s * RPT
    for k in range(RPT // CH2):
        pltpu.sync_copy(agg_sh.at[pl.ds(s * RPT + k * CH2, CH2)],
                        out_hbm.at[pl.ds(base + k * CH2, CH2)])


_edge_kernel = functools.partial(
    pl.kernel,
    out_type=jax.ShapeDtypeStruct((NC * N_PAD, D), jnp.float32),
    mesh=_MESH,
    scratch_types=[
        pltpu.VMEM((BLK, CH2), jnp.int32),
        pltpu.VMEM((BLK, CH2), jnp.int32),
        pltpu.VMEM((CH2, D), jnp.float32),
        pltpu.VMEM((CH2, D), jnp.float32),
        pltpu.VMEM_SHARED((N_PAD, D), jnp.float32),
        pltpu.SemaphoreType.DMA,
        pltpu.SemaphoreType.DMA,
        pltpu.SemaphoreType.DMA,
        pltpu.SemaphoreType.DMA,
    ],
)(_edge_body)


def _norm(deg):
    return jnp.where(deg > 0, lax.rsqrt(jnp.maximum(deg, 1.0)), 0.0)


def _sum_parts(agg_ref):
    return agg_ref[0:N_PAD, :] + agg_ref[N_PAD:, :]


def _tc_in_body(x_ref, dego_ref, w_ref, o_ref):
    h = x_ref[...] * _norm(dego_ref[...])
    o_ref[...] = jnp.dot(h, w_ref[...], preferred_element_type=jnp.float32)


def _tc_mid_body(agg_ref, degi_ref, dego_ref, b_ref, w_ref, o_ref):
    agg = _sum_parts(agg_ref)
    h = jnp.maximum(agg * _norm(degi_ref[...]) + b_ref[...], 0.0)
    h = h * _norm(dego_ref[...])
    o_ref[...] = jnp.dot(h, w_ref[...], preferred_element_type=jnp.float32)


def _tc_out_body(agg_ref, degi_ref, b_ref, o_ref):
    o_ref[...] = _sum_parts(agg_ref) * _norm(degi_ref[...]) + b_ref[...]


def _tc_in(x, dego, w):
    return pl.pallas_call(
        _tc_in_body,
        out_shape=jax.ShapeDtypeStruct((N_PAD, D), jnp.float32),
    )(x, dego, w)


def _tc_mid(agg, degi, dego, b, w):
    return pl.pallas_call(
        _tc_mid_body,
        out_shape=jax.ShapeDtypeStruct((N_PAD, D), jnp.float32),
    )(agg, degi, dego, b, w)


def _tc_out(agg, degi, b):
    return pl.pallas_call(
        _tc_out_body,
        out_shape=jax.ShapeDtypeStruct((N_PAD, D), jnp.float32),
    )(agg, degi, b)


def kernel(x, edge_index, W1, b1, W2, b2, W3, b3):
    src = edge_index[0].astype(jnp.int32)
    dst = edge_index[1].astype(jnp.int32)
    pad = TRASH + jnp.arange(E_PAD - E, dtype=jnp.int32) % (N_PAD - N_NODES)
    src_p = jnp.concatenate([src, pad])
    dst_p = jnp.concatenate([dst, pad])
    srcm = src_p.reshape(-1, CH2)
    dstm = dst_p.reshape(-1, CH2)
    iota = jnp.arange(DEG_ROWS, dtype=jnp.int32).reshape(1, DEG_ROWS)
    zeros = jnp.zeros((128, 128), jnp.float32)

    dego_f, degi_f = _deg_kernel(src_p, dst_p, iota, zeros[:DEG_ROWS])
    dego = dego_f.reshape(NC, -1).sum(0).reshape(N_PAD, 1)
    degi = degi_f.reshape(NC, -1).sum(0).reshape(N_PAD, 1)

    x_pad = jnp.pad(x, ((0, N_PAD - N_NODES), (0, 0)))
    b1r, b2r, b3r = (b.reshape(1, D) for b in (b1, b2, b3))

    h = _tc_in(x_pad, dego, W1)
    agg = _edge_kernel(h, srcm, dstm, zeros)
    h = _tc_mid(agg, degi, dego, b1r, W2)
    agg = _edge_kernel(h, srcm, dstm, zeros)
    h = _tc_mid(agg, degi, dego, b2r, W3)
    agg = _edge_kernel(h, srcm, dstm, zeros)
    return _tc_out(agg, degi, b3r)[:N_NODES]


# speedup vs baseline: 1.0229x; 1.0229x over previous
import functools

import jax
import jax.numpy as jnp
from jax import lax
from jax.experimental import pallas as pl
from jax.experimental.pallas import tpu as pltpu
from jax.experimental.pallas import tpu_sc as plsc

N_NODES = 10000
D = 128
NC, NS = 2, 16
NW = NC * NS
N_PAD = 10240
TRASH = 10000
E = 320000
CH = 128
EROWS = 2560
CPT = EROWS // NW
E_PAD = EROWS * CH
RPT = N_PAD // NS
DEG_ROWS = N_PAD // 128
DEG_RPT = 8

_MESH = plsc.VectorSubcoreMesh(core_axis_name="c", subcore_axis_name="s")


E_PT = E_PAD // NW


def _deg_body(srcv, dstv, iota_hbm, zeros_hbm, out_o, out_i,
              src_v, dst_v, ho, hi, idx_v, tmp_v, sho, shi, sem1, sem2):
    c = lax.axis_index("c")
    s = lax.axis_index("s")
    wid = s * NC + c
    pltpu.sync_copy(zeros_hbm, ho)
    pltpu.sync_copy(zeros_hbm, hi)
    pltpu.sync_copy(zeros_hbm.at[pl.ds(0, DEG_RPT)], tmp_v)

    @pl.when(s < DEG_ROWS // DEG_RPT)
    def _zero_shared():
        pltpu.sync_copy(tmp_v, sho.at[pl.ds(s * DEG_RPT, DEG_RPT)])
        pltpu.sync_copy(tmp_v, shi.at[pl.ds(s * DEG_RPT, DEG_RPT)])

    pltpu.sync_copy(srcv.at[pl.ds(wid * E_PT, E_PT)], src_v)
    pltpu.sync_copy(dstv.at[pl.ds(wid * E_PT, E_PT)], dst_v)
    pltpu.sync_copy(iota_hbm, idx_v)
    ones = jnp.ones((16,), jnp.float32)

    def body(k, carry):
        s16 = src_v[pl.ds(k * 16, 16)]
        d16 = dst_v[pl.ds(k * 16, 16)]
        plsc.addupdate_scatter(ho, [s16 >> 7, s16 & 127], ones)
        plsc.addupdate_scatter(hi, [d16 >> 7, d16 & 127], ones)
        return carry

    lax.fori_loop(0, E_PT // 16, body, 0)
    plsc.subcore_barrier()
    pltpu.async_copy(ho, sho.at[idx_v.at[0]], sem1, add=True).wait()
    pltpu.async_copy(hi, shi.at[idx_v.at[0]], sem2, add=True).wait()
    plsc.subcore_barrier()

    @pl.when(s < DEG_ROWS // DEG_RPT)
    def _writeback():
        base = c * DEG_ROWS + s * DEG_RPT
        pltpu.sync_copy(sho.at[pl.ds(s * DEG_RPT, DEG_RPT)], tmp_v)
        pltpu.sync_copy(tmp_v, out_o.at[pl.ds(base, DEG_RPT)])
        pltpu.sync_copy(shi.at[pl.ds(s * DEG_RPT, DEG_RPT)], tmp_v)
        pltpu.sync_copy(tmp_v, out_i.at[pl.ds(base, DEG_RPT)])


_deg_kernel = functools.partial(
    pl.kernel,
    out_type=(jax.ShapeDtypeStruct((NC * DEG_ROWS, 128), jnp.float32),
              jax.ShapeDtypeStruct((NC * DEG_ROWS, 128), jnp.float32)),
    mesh=_MESH,
    scratch_types=[
        pltpu.VMEM((E_PT,), jnp.int32),
        pltpu.VMEM((E_PT,), jnp.int32),
        pltpu.VMEM((DEG_ROWS, 128), jnp.float32),
        pltpu.VMEM((DEG_ROWS, 128), jnp.float32),
        pltpu.VMEM((1, DEG_ROWS), jnp.int32),
        pltpu.VMEM((DEG_RPT, 128), jnp.float32),
        pltpu.VMEM_SHARED((DEG_ROWS, 128), jnp.float32),
        pltpu.VMEM_SHARED((DEG_ROWS, 128), jnp.float32),
        pltpu.SemaphoreType.DMA,
        pltpu.SemaphoreType.DMA,
    ],
    compiler_params=pltpu.CompilerParams(needs_layout_passes=False),
)(_deg_body)


CH2 = 128
CROWS_PT = E_PAD // CH2 // NW
BLK = 16


def _edge_body(h_hbm, srcm, dstm, zeros_hbm, out_hbm,
               src_v, dst_v, rows0_v, rows1_v, agg_sh,
               gsem0, gsem1, ssem0, ssem1):
    c = lax.axis_index("c")
    s = lax.axis_index("s")
    wid = s * NC + c
    rows = (rows0_v, rows1_v)
    gsems = (gsem0, gsem1)
    ssems = (ssem0, ssem1)
    pltpu.sync_copy(zeros_hbm, rows0_v)
    for k in range(RPT // CH2):
        pltpu.sync_copy(rows0_v, agg_sh.at[pl.ds(s * RPT + k * CH2, CH2)])
    plsc.subcore_barrier()
    for p in range(CROWS_PT // BLK):
        base_row = wid * CROWS_PT + p * BLK
        pltpu.sync_copy(srcm.at[pl.ds(base_row, BLK)], src_v)
        pltpu.sync_copy(dstm.at[pl.ds(base_row, BLK)], dst_v)
        g = pltpu.async_copy(h_hbm.at[src_v.at[0]], rows[0], gsems[0])
        sd = [None, None]
        for j in range(BLK):
            b = j & 1
            nb = b ^ 1
            g.wait()
            if j + 1 < BLK:
                if sd[nb] is not None:
                    sd[nb].wait()
                g = pltpu.async_copy(h_hbm.at[src_v.at[j + 1]], rows[nb],
                                     gsems[nb])
            sd[b] = pltpu.async_copy(rows[b], agg_sh.at[dst_v.at[j]],
                                     ssems[b], add=True)
        sd[0].wait()
        sd[1].wait()
    plsc.subcore_barrier()
    base = c * N_PAD + s * RPT
    for k in range(RPT // CH2):
        pltpu.sync_copy(agg_sh.at[pl.ds(s * RPT + k * CH2, CH2)], rows0_v)
        pltpu.sync_copy(rows0_v, out_hbm.at[pl.ds(base + k * CH2, CH2)])


_edge_kernel = functools.partial(
    pl.kernel,
    out_type=jax.ShapeDtypeStruct((NC * N_PAD, D), jnp.float32),
    mesh=_MESH,
    scratch_types=[
        pltpu.VMEM((BLK, CH2), jnp.int32),
        pltpu.VMEM((BLK, CH2), jnp.int32),
        pltpu.VMEM((CH2, D), jnp.float32),
        pltpu.VMEM((CH2, D), jnp.float32),
        pltpu.VMEM_SHARED((N_PAD, D), jnp.float32),
        pltpu.SemaphoreType.DMA,
        pltpu.SemaphoreType.DMA,
        pltpu.SemaphoreType.DMA,
        pltpu.SemaphoreType.DMA,
    ],
)(_edge_body)


def _norm(deg):
    return jnp.where(deg > 0, lax.rsqrt(jnp.maximum(deg, 1.0)), 0.0)


def _sum_parts(agg_ref):
    return agg_ref[0:N_PAD, :] + agg_ref[N_PAD:, :]


def _tc_in_body(x_ref, w_ref, o_ref):
    o_ref[...] = jnp.dot(x_ref[...], w_ref[...],
                         preferred_element_type=jnp.float32)


def _tc_scale_body(xw_ref, dego_ref, o_ref):
    o_ref[...] = xw_ref[...] * _norm(dego_ref[...])


def _tc_mid_body(agg_ref, degi_ref, dego_ref, b_ref, w_ref, o_ref):
    agg = _sum_parts(agg_ref)
    h = jnp.maximum(agg * _norm(degi_ref[...]) + b_ref[...], 0.0)
    h = h * _norm(dego_ref[...])
    o_ref[...] = jnp.dot(h, w_ref[...], preferred_element_type=jnp.float32)


def _tc_out_body(agg_ref, degi_ref, b_ref, o_ref):
    o_ref[...] = _sum_parts(agg_ref) * _norm(degi_ref[...]) + b_ref[...]


def _tc_in(x, w):
    return pl.pallas_call(
        _tc_in_body,
        out_shape=jax.ShapeDtypeStruct((N_PAD, D), jnp.float32),
    )(x, w)


def _tc_scale(xw, dego):
    return pl.pallas_call(
        _tc_scale_body,
        out_shape=jax.ShapeDtypeStruct((N_PAD, D), jnp.float32),
    )(xw, dego)


def _tc_mid(agg, degi, dego, b, w):
    return pl.pallas_call(
        _tc_mid_body,
        out_shape=jax.ShapeDtypeStruct((N_PAD, D), jnp.float32),
    )(agg, degi, dego, b, w)


def _tc_out(agg, degi, b):
    return pl.pallas_call(
        _tc_out_body,
        out_shape=jax.ShapeDtypeStruct((N_PAD, D), jnp.float32),
    )(agg, degi, b)


def kernel(x, edge_index, W1, b1, W2, b2, W3, b3):
    src = edge_index[0].astype(jnp.int32)
    dst = edge_index[1].astype(jnp.int32)
    pad = TRASH + jnp.arange(E_PAD - E, dtype=jnp.int32) % (N_PAD - N_NODES)
    src_p = jnp.concatenate([src, pad])
    dst_p = jnp.concatenate([dst, pad])
    srcm = src_p.reshape(-1, CH2)
    dstm = dst_p.reshape(-1, CH2)
    iota = jnp.arange(DEG_ROWS, dtype=jnp.int32).reshape(1, DEG_ROWS)
    zeros = jnp.zeros((128, 128), jnp.float32)

    x_pad = jnp.pad(x, ((0, N_PAD - N_NODES), (0, 0)))
    b1r, b2r, b3r = (b.reshape(1, D) for b in (b1, b2, b3))

    xw = _tc_in(x_pad, W1)
    dego_f, degi_f = _deg_kernel(src_p, dst_p, iota, zeros[:DEG_ROWS])
    dego = dego_f.reshape(NC, -1).sum(0).reshape(N_PAD, 1)
    degi = degi_f.reshape(NC, -1).sum(0).reshape(N_PAD, 1)

    h = _tc_scale(xw, dego)
    agg = _edge_kernel(h, srcm, dstm, zeros)
    h = _tc_mid(agg, degi, dego, b1r, W2)
    agg = _edge_kernel(h, srcm, dstm, zeros)
    h = _tc_mid(agg, degi, dego, b2r, W3)
    agg = _edge_kernel(h, srcm, dstm, zeros)
    return _tc_out(agg, degi, b3r)[:N_NODES]


# speedup vs baseline: 1.0284x; 1.0054x over previous
import functools

import jax
import jax.numpy as jnp
from jax import lax
from jax.experimental import pallas as pl
from jax.experimental.pallas import tpu as pltpu
from jax.experimental.pallas import tpu_sc as plsc

N_NODES = 10000
D = 128
NC, NS = 2, 16
NW = NC * NS
N_PAD = 10240
TRASH = 10000
E = 320000
CH = 128
EROWS = 2560
CPT = EROWS // NW
E_PAD = EROWS * CH
RPT = N_PAD // NS
DEG_ROWS = N_PAD // 128
DEG_RPT = 8

_MESH = plsc.VectorSubcoreMesh(core_axis_name="c", subcore_axis_name="s")


E_PT = E_PAD // NW


def _deg_body(srcv, dstv, iota_hbm, zeros_hbm, out_o, out_i,
              src_v, dst_v, ho, hi, idx_v, tmp_v, sho, shi, sem1, sem2):
    c = lax.axis_index("c")
    s = lax.axis_index("s")
    wid = s * NC + c
    pltpu.sync_copy(zeros_hbm, ho)
    pltpu.sync_copy(zeros_hbm, hi)
    pltpu.sync_copy(zeros_hbm.at[pl.ds(0, DEG_RPT)], tmp_v)

    @pl.when(s < DEG_ROWS // DEG_RPT)
    def _zero_shared():
        pltpu.sync_copy(tmp_v, sho.at[pl.ds(s * DEG_RPT, DEG_RPT)])
        pltpu.sync_copy(tmp_v, shi.at[pl.ds(s * DEG_RPT, DEG_RPT)])

    pltpu.sync_copy(srcv.at[pl.ds(wid * E_PT, E_PT)], src_v)
    pltpu.sync_copy(dstv.at[pl.ds(wid * E_PT, E_PT)], dst_v)
    pltpu.sync_copy(iota_hbm, idx_v)
    ones = jnp.ones((16,), jnp.float32)

    def body(k, carry):
        s16 = src_v[pl.ds(k * 16, 16)]
        d16 = dst_v[pl.ds(k * 16, 16)]
        plsc.addupdate_scatter(ho, [s16 >> 7, s16 & 127], ones)
        plsc.addupdate_scatter(hi, [d16 >> 7, d16 & 127], ones)
        return carry

    lax.fori_loop(0, E_PT // 16, body, 0)
    plsc.subcore_barrier()
    pltpu.async_copy(ho, sho.at[idx_v.at[0]], sem1, add=True).wait()
    pltpu.async_copy(hi, shi.at[idx_v.at[0]], sem2, add=True).wait()
    plsc.subcore_barrier()

    @pl.when(s < DEG_ROWS // DEG_RPT)
    def _writeback():
        base = c * DEG_ROWS + s * DEG_RPT
        pltpu.sync_copy(sho.at[pl.ds(s * DEG_RPT, DEG_RPT)], tmp_v)
        pltpu.sync_copy(tmp_v, out_o.at[pl.ds(base, DEG_RPT)])
        pltpu.sync_copy(shi.at[pl.ds(s * DEG_RPT, DEG_RPT)], tmp_v)
        pltpu.sync_copy(tmp_v, out_i.at[pl.ds(base, DEG_RPT)])


_deg_kernel = functools.partial(
    pl.kernel,
    out_type=(jax.ShapeDtypeStruct((NC * DEG_ROWS, 128), jnp.float32),
              jax.ShapeDtypeStruct((NC * DEG_ROWS, 128), jnp.float32)),
    mesh=_MESH,
    scratch_types=[
        pltpu.VMEM((E_PT,), jnp.int32),
        pltpu.VMEM((E_PT,), jnp.int32),
        pltpu.VMEM((DEG_ROWS, 128), jnp.float32),
        pltpu.VMEM((DEG_ROWS, 128), jnp.float32),
        pltpu.VMEM((1, DEG_ROWS), jnp.int32),
        pltpu.VMEM((DEG_RPT, 128), jnp.float32),
        pltpu.VMEM_SHARED((DEG_ROWS, 128), jnp.float32),
        pltpu.VMEM_SHARED((DEG_ROWS, 128), jnp.float32),
        pltpu.SemaphoreType.DMA,
        pltpu.SemaphoreType.DMA,
    ],
    compiler_params=pltpu.CompilerParams(needs_layout_passes=False),
)(_deg_body)


CH2 = 128
CROWS_PT = E_PAD // CH2 // NW
BLK = 16


def _edge_body(h_hbm, srcm, dstm, zeros_hbm, out_hbm,
               src_v, dst_v, rows0_v, rows1_v, agg_sh,
               gsem0, gsem1, ssem0, ssem1):
    c = lax.axis_index("c")
    s = lax.axis_index("s")
    wid = s * NC + c
    rows = (rows0_v, rows1_v)
    gsems = (gsem0, gsem1)
    ssems = (ssem0, ssem1)
    pltpu.sync_copy(zeros_hbm, rows0_v)
    for k in range(RPT // CH2):
        pltpu.sync_copy(rows0_v, agg_sh.at[pl.ds(s * RPT + k * CH2, CH2)])
    plsc.subcore_barrier()
    for p in range(CROWS_PT // BLK):
        base_row = wid * CROWS_PT + p * BLK
        pltpu.sync_copy(srcm.at[pl.ds(base_row, BLK)], src_v)
        pltpu.sync_copy(dstm.at[pl.ds(base_row, BLK)], dst_v)
        g = pltpu.async_copy(h_hbm.at[src_v.at[0]], rows[0], gsems[0])
        sd = [None, None]
        for j in range(BLK):
            b = j & 1
            nb = b ^ 1
            g.wait()
            sd[b] = pltpu.async_copy(rows[b], agg_sh.at[dst_v.at[j]],
                                     ssems[b], add=True)
            if j + 1 < BLK:
                if sd[nb] is not None:
                    sd[nb].wait()
                g = pltpu.async_copy(h_hbm.at[src_v.at[j + 1]], rows[nb],
                                     gsems[nb])
        sd[0].wait()
        sd[1].wait()
    plsc.subcore_barrier()
    base = c * N_PAD + s * RPT
    for k in range(RPT // CH2):
        pltpu.sync_copy(agg_sh.at[pl.ds(s * RPT + k * CH2, CH2)], rows0_v)
        pltpu.sync_copy(rows0_v, out_hbm.at[pl.ds(base + k * CH2, CH2)])


_edge_kernel = functools.partial(
    pl.kernel,
    out_type=jax.ShapeDtypeStruct((NC * N_PAD, D), jnp.float32),
    mesh=_MESH,
    scratch_types=[
        pltpu.VMEM((BLK, CH2), jnp.int32),
        pltpu.VMEM((BLK, CH2), jnp.int32),
        pltpu.VMEM((CH2, D), jnp.float32),
        pltpu.VMEM((CH2, D), jnp.float32),
        pltpu.VMEM_SHARED((N_PAD, D), jnp.float32),
        pltpu.SemaphoreType.DMA,
        pltpu.SemaphoreType.DMA,
        pltpu.SemaphoreType.DMA,
        pltpu.SemaphoreType.DMA,
    ],
)(_edge_body)


def _norm(deg):
    return jnp.where(deg > 0, lax.rsqrt(jnp.maximum(deg, 1.0)), 0.0)


def _sum_parts(agg_ref):
    return agg_ref[0:N_PAD, :] + agg_ref[N_PAD:, :]


def _tc_in_body(x_ref, w_ref, o_ref):
    o_ref[...] = jnp.dot(x_ref[...], w_ref[...],
                         preferred_element_type=jnp.float32)


def _tc_scale_body(xw_ref, dego_ref, o_ref):
    o_ref[...] = xw_ref[...] * _norm(dego_ref[...])


def _tc_mid_body(agg_ref, degi_ref, dego_ref, b_ref, w_ref, o_ref):
    agg = _sum_parts(agg_ref)
    h = jnp.maximum(agg * _norm(degi_ref[...]) + b_ref[...], 0.0)
    h = h * _norm(dego_ref[...])
    o_ref[...] = jnp.dot(h, w_ref[...], preferred_element_type=jnp.float32)


def _tc_out_body(agg_ref, degi_ref, b_ref, o_ref):
    o_ref[...] = _sum_parts(agg_ref) * _norm(degi_ref[...]) + b_ref[...]


def _tc_in(x, w):
    return pl.pallas_call(
        _tc_in_body,
        out_shape=jax.ShapeDtypeStruct((N_PAD, D), jnp.float32),
    )(x, w)


def _tc_scale(xw, dego):
    return pl.pallas_call(
        _tc_scale_body,
        out_shape=jax.ShapeDtypeStruct((N_PAD, D), jnp.float32),
    )(xw, dego)


def _tc_mid(agg, degi, dego, b, w):
    return pl.pallas_call(
        _tc_mid_body,
        out_shape=jax.ShapeDtypeStruct((N_PAD, D), jnp.float32),
    )(agg, degi, dego, b, w)


def _tc_out(agg, degi, b):
    return pl.pallas_call(
        _tc_out_body,
        out_shape=jax.ShapeDtypeStruct((N_PAD, D), jnp.float32),
    )(agg, degi, b)


def kernel(x, edge_index, W1, b1, W2, b2, W3, b3):
    src = edge_index[0].astype(jnp.int32)
    dst = edge_index[1].astype(jnp.int32)
    pad = TRASH + jnp.arange(E_PAD - E, dtype=jnp.int32) % (N_PAD - N_NODES)
    src_p = jnp.concatenate([src, pad])
    dst_p = jnp.concatenate([dst, pad])
    srcm = src_p.reshape(-1, CH2)
    dstm = dst_p.reshape(-1, CH2)
    iota = jnp.arange(DEG_ROWS, dtype=jnp.int32).reshape(1, DEG_ROWS)
    zeros = jnp.zeros((128, 128), jnp.float32)

    dego_f, degi_f = _deg_kernel(src_p, dst_p, iota, zeros[:DEG_ROWS])
    dego = dego_f.reshape(NC, -1).sum(0).reshape(N_PAD, 1)
    degi = degi_f.reshape(NC, -1).sum(0).reshape(N_PAD, 1)

    x_pad = jnp.pad(x, ((0, N_PAD - N_NODES), (0, 0)))
    b1r, b2r, b3r = (b.reshape(1, D) for b in (b1, b2, b3))

    h = _tc_scale(_tc_in(x_pad, W1), dego)
    agg = _edge_kernel(h, srcm, dstm, zeros)
    h = _tc_mid(agg, degi, dego, b1r, W2)
    agg = _edge_kernel(h, srcm, dstm, zeros)
    h = _tc_mid(agg, degi, dego, b2r, W3)
    agg = _edge_kernel(h, srcm, dstm, zeros)
    return _tc_out(agg, degi, b3r)[:N_NODES]


# speedup vs baseline: 1.0357x; 1.0070x over previous
import functools

import jax
import jax.numpy as jnp
from jax import lax
from jax.experimental import pallas as pl
from jax.experimental.pallas import tpu as pltpu
from jax.experimental.pallas import tpu_sc as plsc

N_NODES = 10000
D = 128
NC, NS = 2, 16
NW = NC * NS
N_PAD = 10240
TRASH = 10000
E = 320000
CH = 128
EROWS = 2560
CPT = EROWS // NW
E_PAD = EROWS * CH
RPT = N_PAD // NS
DEG_ROWS = N_PAD // 128
DEG_RPT = 8

_MESH = plsc.VectorSubcoreMesh(core_axis_name="c", subcore_axis_name="s")


E_PT = E_PAD // NW


def _deg_body(srcv, dstv, iota_hbm, zeros_hbm, out_o, out_i,
              src_v, dst_v, ho, hi, idx_v, tmp_v, sho, shi, sem1, sem2):
    c = lax.axis_index("c")
    s = lax.axis_index("s")
    wid = s * NC + c
    pltpu.sync_copy(zeros_hbm, ho)
    pltpu.sync_copy(zeros_hbm, hi)
    pltpu.sync_copy(zeros_hbm.at[pl.ds(0, DEG_RPT)], tmp_v)

    @pl.when(s < DEG_ROWS // DEG_RPT)
    def _zero_shared():
        pltpu.sync_copy(tmp_v, sho.at[pl.ds(s * DEG_RPT, DEG_RPT)])
        pltpu.sync_copy(tmp_v, shi.at[pl.ds(s * DEG_RPT, DEG_RPT)])

    pltpu.sync_copy(srcv.at[pl.ds(wid * E_PT, E_PT)], src_v)
    pltpu.sync_copy(dstv.at[pl.ds(wid * E_PT, E_PT)], dst_v)
    pltpu.sync_copy(iota_hbm, idx_v)
    ones = jnp.ones((16,), jnp.float32)

    def body(k, carry):
        s16 = src_v[pl.ds(k * 16, 16)]
        d16 = dst_v[pl.ds(k * 16, 16)]
        plsc.addupdate_scatter(ho, [s16 >> 7, s16 & 127], ones)
        plsc.addupdate_scatter(hi, [d16 >> 7, d16 & 127], ones)
        return carry

    lax.fori_loop(0, E_PT // 16, body, 0)
    plsc.subcore_barrier()
    pltpu.async_copy(ho, sho.at[idx_v.at[0]], sem1, add=True).wait()
    pltpu.async_copy(hi, shi.at[idx_v.at[0]], sem2, add=True).wait()
    plsc.subcore_barrier()

    @pl.when(s < DEG_ROWS // DEG_RPT)
    def _writeback():
        base = c * DEG_ROWS + s * DEG_RPT
        pltpu.sync_copy(sho.at[pl.ds(s * DEG_RPT, DEG_RPT)], tmp_v)
        pltpu.sync_copy(tmp_v, out_o.at[pl.ds(base, DEG_RPT)])
        pltpu.sync_copy(shi.at[pl.ds(s * DEG_RPT, DEG_RPT)], tmp_v)
        pltpu.sync_copy(tmp_v, out_i.at[pl.ds(base, DEG_RPT)])


_deg_kernel = functools.partial(
    pl.kernel,
    out_type=(jax.ShapeDtypeStruct((NC * DEG_ROWS, 128), jnp.float32),
              jax.ShapeDtypeStruct((NC * DEG_ROWS, 128), jnp.float32)),
    mesh=_MESH,
    scratch_types=[
        pltpu.VMEM((E_PT,), jnp.int32),
        pltpu.VMEM((E_PT,), jnp.int32),
        pltpu.VMEM((DEG_ROWS, 128), jnp.float32),
        pltpu.VMEM((DEG_ROWS, 128), jnp.float32),
        pltpu.VMEM((1, DEG_ROWS), jnp.int32),
        pltpu.VMEM((DEG_RPT, 128), jnp.float32),
        pltpu.VMEM_SHARED((DEG_ROWS, 128), jnp.float32),
        pltpu.VMEM_SHARED((DEG_ROWS, 128), jnp.float32),
        pltpu.SemaphoreType.DMA,
        pltpu.SemaphoreType.DMA,
    ],
    compiler_params=pltpu.CompilerParams(needs_layout_passes=False),
)(_deg_body)


CH2 = 128
CROWS_PT = E_PAD // CH2 // NW
BLK = 16


def _edge_body(h_hbm, srcm, dstm, zeros_hbm, out_hbm,
               src_v, dst_v, rows0_v, rows1_v, agg_sh,
               gsem0, gsem1, ssem0, ssem1):
    c = lax.axis_index("c")
    s = lax.axis_index("s")
    wid = s * NC + c
    rows = (rows0_v, rows1_v)
    gsems = (gsem0, gsem1)
    ssems = (ssem0, ssem1)
    pltpu.sync_copy(zeros_hbm, rows0_v)
    for k in range(RPT // CH2):
        pltpu.sync_copy(rows0_v, agg_sh.at[pl.ds(s * RPT + k * CH2, CH2)])
    plsc.subcore_barrier()
    for p in range(CROWS_PT // BLK):
        base_row = wid * CROWS_PT + p * BLK
        pltpu.sync_copy(srcm.at[pl.ds(base_row, BLK)], src_v)
        pltpu.sync_copy(dstm.at[pl.ds(base_row, BLK)], dst_v)
        g = pltpu.async_copy(h_hbm.at[src_v.at[0]], rows[0], gsems[0])
        sd = [None, None]
        for j in range(BLK):
            b = j & 1
            nb = b ^ 1
            g.wait()
            sd[b] = pltpu.async_copy(rows[b], agg_sh.at[dst_v.at[j]],
                                     ssems[b], add=True)
            if j + 1 < BLK:
                if sd[nb] is not None:
                    sd[nb].wait()
                g = pltpu.async_copy(h_hbm.at[src_v.at[j + 1]], rows[nb],
                                     gsems[nb])
        sd[0].wait()
        sd[1].wait()
    plsc.subcore_barrier()
    base = c * N_PAD + s * RPT
    for k in range(RPT // CH2):
        pltpu.sync_copy(agg_sh.at[pl.ds(s * RPT + k * CH2, CH2)], rows0_v)
        pltpu.sync_copy(rows0_v, out_hbm.at[pl.ds(base + k * CH2, CH2)])


_edge_kernel = functools.partial(
    pl.kernel,
    out_type=jax.ShapeDtypeStruct((NC * N_PAD, D), jnp.float32),
    mesh=_MESH,
    scratch_types=[
        pltpu.VMEM((BLK, CH2), jnp.int32),
        pltpu.VMEM((BLK, CH2), jnp.int32),
        pltpu.VMEM((CH2, D), jnp.float32),
        pltpu.VMEM((CH2, D), jnp.float32),
        pltpu.VMEM_SHARED((N_PAD, D), jnp.float32),
        pltpu.SemaphoreType.DMA,
        pltpu.SemaphoreType.DMA,
        pltpu.SemaphoreType.DMA,
        pltpu.SemaphoreType.DMA,
    ],
)(_edge_body)


def _norm(deg):
    return jnp.where(deg > 0, lax.rsqrt(jnp.maximum(deg, 1.0)), 0.0)


def _sum_parts(agg_ref):
    return agg_ref[0:N_PAD, :] + agg_ref[N_PAD:, :]


def _tc_in_body(x_ref, dego_ref, w_ref, o_ref):
    h = x_ref[...] * _norm(dego_ref[...])
    o_ref[...] = jnp.dot(h, w_ref[...], preferred_element_type=jnp.float32)


def _tc_mid_body(agg_ref, degi_ref, dego_ref, b_ref, w_ref, o_ref):
    agg = _sum_parts(agg_ref)
    h = jnp.maximum(agg * _norm(degi_ref[...]) + b_ref[...], 0.0)
    h = h * _norm(dego_ref[...])
    o_ref[...] = jnp.dot(h, w_ref[...], preferred_element_type=jnp.float32)


def _tc_out_body(agg_ref, degi_ref, b_ref, o_ref):
    o_ref[...] = _sum_parts(agg_ref) * _norm(degi_ref[...]) + b_ref[...]


def _tc_in(x, dego, w):
    return pl.pallas_call(
        _tc_in_body,
        out_shape=jax.ShapeDtypeStruct((N_PAD, D), jnp.float32),
    )(x, dego, w)


def _tc_mid(agg, degi, dego, b, w):
    return pl.pallas_call(
        _tc_mid_body,
        out_shape=jax.ShapeDtypeStruct((N_PAD, D), jnp.float32),
    )(agg, degi, dego, b, w)


def _tc_out(agg, degi, b):
    return pl.pallas_call(
        _tc_out_body,
        out_shape=jax.ShapeDtypeStruct((N_PAD, D), jnp.float32),
    )(agg, degi, b)


def kernel(x, edge_index, W1, b1, W2, b2, W3, b3):
    src = edge_index[0].astype(jnp.int32)
    dst = edge_index[1].astype(jnp.int32)
    pad = TRASH + jnp.arange(E_PAD - E, dtype=jnp.int32) % (N_PAD - N_NODES)
    src_p = jnp.concatenate([src, pad])
    dst_p = jnp.concatenate([dst, pad])
    srcm = src_p.reshape(-1, CH2)
    dstm = dst_p.reshape(-1, CH2)
    iota = jnp.arange(DEG_ROWS, dtype=jnp.int32).reshape(1, DEG_ROWS)
    zeros = jnp.zeros((128, 128), jnp.float32)

    dego_f, degi_f = _deg_kernel(src_p, dst_p, iota, zeros[:DEG_ROWS])
    dego = dego_f.reshape(NC, -1).sum(0).reshape(N_PAD, 1)
    degi = degi_f.reshape(NC, -1).sum(0).reshape(N_PAD, 1)

    x_pad = jnp.pad(x, ((0, N_PAD - N_NODES), (0, 0)))
    b1r, b2r, b3r = (b.reshape(1, D) for b in (b1, b2, b3))

    h = _tc_in(x_pad, dego, W1)
    agg = _edge_kernel(h, srcm, dstm, zeros)
    h = _tc_mid(agg, degi, dego, b1r, W2)
    agg = _edge_kernel(h, srcm, dstm, zeros)
    h = _tc_mid(agg, degi, dego, b2r, W3)
    agg = _edge_kernel(h, srcm, dstm, zeros)
    return _tc_out(agg, degi, b3r)[:N_NODES]


# speedup vs baseline: 1.0839x; 1.0466x over previous
import functools

import jax
import jax.numpy as jnp
from jax import lax
from jax.experimental import pallas as pl
from jax.experimental.pallas import tpu as pltpu
from jax.experimental.pallas import tpu_sc as plsc

N_NODES = 10000
D = 128
NC, NS = 2, 16
NW = NC * NS
N_PAD = 10240
TRASH = 10000
E = 320000
CH = 128
EROWS = 2560
CPT = EROWS // NW
E_PAD = EROWS * CH
RPT = N_PAD // NS
DEG_ROWS = N_PAD // 128
DEG_RPT = 8

_MESH = plsc.VectorSubcoreMesh(core_axis_name="c", subcore_axis_name="s")


E_PT = E_PAD // NW


def _deg_body(srcv, dstv, iota_hbm, zeros_hbm, out_o, out_i,
              src_v, dst_v, ho, hi, idx_v, tmp_v, sho, shi, sem1, sem2):
    c = lax.axis_index("c")
    s = lax.axis_index("s")
    wid = s * NC + c
    pltpu.sync_copy(zeros_hbm, ho)
    pltpu.sync_copy(zeros_hbm, hi)
    pltpu.sync_copy(zeros_hbm.at[pl.ds(0, DEG_RPT)], tmp_v)

    @pl.when(s < DEG_ROWS // DEG_RPT)
    def _zero_shared():
        pltpu.sync_copy(tmp_v, sho.at[pl.ds(s * DEG_RPT, DEG_RPT)])
        pltpu.sync_copy(tmp_v, shi.at[pl.ds(s * DEG_RPT, DEG_RPT)])

    pltpu.sync_copy(srcv.at[pl.ds(wid * E_PT, E_PT)], src_v)
    pltpu.sync_copy(dstv.at[pl.ds(wid * E_PT, E_PT)], dst_v)
    pltpu.sync_copy(iota_hbm, idx_v)
    ones = jnp.ones((16,), jnp.float32)

    def body(k, carry):
        s16 = src_v[pl.ds(k * 16, 16)]
        d16 = dst_v[pl.ds(k * 16, 16)]
        plsc.addupdate_scatter(ho, [s16 >> 7, s16 & 127], ones)
        plsc.addupdate_scatter(hi, [d16 >> 7, d16 & 127], ones)
        return carry

    lax.fori_loop(0, E_PT // 16, body, 0)
    plsc.subcore_barrier()
    pltpu.async_copy(ho, sho.at[idx_v.at[0]], sem1, add=True).wait()
    pltpu.async_copy(hi, shi.at[idx_v.at[0]], sem2, add=True).wait()
    plsc.subcore_barrier()

    @pl.when(s < DEG_ROWS // DEG_RPT)
    def _writeback():
        base = c * DEG_ROWS + s * DEG_RPT
        pltpu.sync_copy(sho.at[pl.ds(s * DEG_RPT, DEG_RPT)], tmp_v)
        pltpu.sync_copy(tmp_v, out_o.at[pl.ds(base, DEG_RPT)])
        pltpu.sync_copy(shi.at[pl.ds(s * DEG_RPT, DEG_RPT)], tmp_v)
        pltpu.sync_copy(tmp_v, out_i.at[pl.ds(base, DEG_RPT)])


_deg_kernel = functools.partial(
    pl.kernel,
    out_type=(jax.ShapeDtypeStruct((NC * DEG_ROWS, 128), jnp.float32),
              jax.ShapeDtypeStruct((NC * DEG_ROWS, 128), jnp.float32)),
    mesh=_MESH,
    scratch_types=[
        pltpu.VMEM((E_PT,), jnp.int32),
        pltpu.VMEM((E_PT,), jnp.int32),
        pltpu.VMEM((DEG_ROWS, 128), jnp.float32),
        pltpu.VMEM((DEG_ROWS, 128), jnp.float32),
        pltpu.VMEM((1, DEG_ROWS), jnp.int32),
        pltpu.VMEM((DEG_RPT, 128), jnp.float32),
        pltpu.VMEM_SHARED((DEG_ROWS, 128), jnp.float32),
        pltpu.VMEM_SHARED((DEG_ROWS, 128), jnp.float32),
        pltpu.SemaphoreType.DMA,
        pltpu.SemaphoreType.DMA,
    ],
    compiler_params=pltpu.CompilerParams(needs_layout_passes=False),
)(_deg_body)


CH2 = 128
CROWS_PT = E_PAD // CH2 // NW
BLK = 16


def _edge_body(h_hbm, srcm, dstm, zeros_hbm, out_hbm,
               src_v, dst_v, rows0_v, rows1_v, agg_sh,
               gsem0, gsem1, ssem0, ssem1, isem):
    c = lax.axis_index("c")
    s = lax.axis_index("s")
    wid = s * NC + c
    rows = (rows0_v, rows1_v)
    gsems = (gsem0, gsem1)
    ssems = (ssem0, ssem1)
    nblk = CROWS_PT // BLK
    pltpu.sync_copy(zeros_hbm, rows0_v)
    zds = [pltpu.async_copy(rows0_v, agg_sh.at[pl.ds(s * RPT + k * CH2, CH2)],
                            gsem0)
           for k in range(RPT // CH2)]
    i_src = pltpu.async_copy(srcm.at[pl.ds(wid * CROWS_PT, BLK)],
                             src_v.at[pl.ds(0, BLK)], isem)
    i_dst = pltpu.async_copy(dstm.at[pl.ds(wid * CROWS_PT, BLK)],
                             dst_v.at[pl.ds(0, BLK)], isem)
    for z in zds:
        z.wait()
    plsc.subcore_barrier()
    for p in range(nblk):
        slot = (p & 1) * BLK
        i_src.wait()
        i_dst.wait()
        if p + 1 < nblk:
            nxt = wid * CROWS_PT + (p + 1) * BLK
            nslot = ((p + 1) & 1) * BLK
            i_src = pltpu.async_copy(srcm.at[pl.ds(nxt, BLK)],
                                     src_v.at[pl.ds(nslot, BLK)], isem)
            i_dst = pltpu.async_copy(dstm.at[pl.ds(nxt, BLK)],
                                     dst_v.at[pl.ds(nslot, BLK)], isem)
        g = pltpu.async_copy(h_hbm.at[src_v.at[slot]], rows[0], gsems[0])
        sd = [None, None]
        for j in range(BLK):
            b = j & 1
            nb = b ^ 1
            g.wait()
            sd[b] = pltpu.async_copy(rows[b], agg_sh.at[dst_v.at[slot + j]],
                                     ssems[b], add=True)
            if j + 1 < BLK:
                if sd[nb] is not None:
                    sd[nb].wait()
                g = pltpu.async_copy(h_hbm.at[src_v.at[slot + j + 1]],
                                     rows[nb], gsems[nb])
        sd[0].wait()
        sd[1].wait()
    plsc.subcore_barrier()
    base = c * N_PAD + s * RPT
    wd = [None, None]
    for k in range(RPT // CH2):
        b = k & 1
        if wd[b] is not None:
            wd[b].wait()
        pltpu.sync_copy(agg_sh.at[pl.ds(s * RPT + k * CH2, CH2)], rows[b])
        wd[b] = pltpu.async_copy(rows[b],
                                 out_hbm.at[pl.ds(base + k * CH2, CH2)],
                                 gsems[b])
    wd[0].wait()
    wd[1].wait()


_edge_kernel = functools.partial(
    pl.kernel,
    out_type=jax.ShapeDtypeStruct((NC * N_PAD, D), jnp.float32),
    mesh=_MESH,
    scratch_types=[
        pltpu.VMEM((2 * BLK, CH2), jnp.int32),
        pltpu.VMEM((2 * BLK, CH2), jnp.int32),
        pltpu.VMEM((CH2, D), jnp.float32),
        pltpu.VMEM((CH2, D), jnp.float32),
        pltpu.VMEM_SHARED((N_PAD, D), jnp.float32),
        pltpu.SemaphoreType.DMA,
        pltpu.SemaphoreType.DMA,
        pltpu.SemaphoreType.DMA,
        pltpu.SemaphoreType.DMA,
        pltpu.SemaphoreType.DMA,
    ],
)(_edge_body)


def _norm(deg):
    return jnp.where(deg > 0, lax.rsqrt(jnp.maximum(deg, 1.0)), 0.0)


def _sum_parts(agg_ref):
    return agg_ref[0:N_PAD, :] + agg_ref[N_PAD:, :]


def _tc_in_body(x_ref, dego_ref, w_ref, o_ref):
    h = x_ref[...] * _norm(dego_ref[...])
    o_ref[...] = jnp.dot(h, w_ref[...], preferred_element_type=jnp.float32)


def _tc_mid_body(agg_ref, degi_ref, dego_ref, b_ref, w_ref, o_ref):
    agg = _sum_parts(agg_ref)
    h = jnp.maximum(agg * _norm(degi_ref[...]) + b_ref[...], 0.0)
    h = h * _norm(dego_ref[...])
    o_ref[...] = jnp.dot(h, w_ref[...], preferred_element_type=jnp.float32)


def _tc_out_body(agg_ref, degi_ref, b_ref, o_ref):
    o_ref[...] = _sum_parts(agg_ref) * _norm(degi_ref[...]) + b_ref[...]


def _tc_in(x, dego, w):
    return pl.pallas_call(
        _tc_in_body,
        out_shape=jax.ShapeDtypeStruct((N_PAD, D), jnp.float32),
    )(x, dego, w)


def _tc_mid(agg, degi, dego, b, w):
    return pl.pallas_call(
        _tc_mid_body,
        out_shape=jax.ShapeDtypeStruct((N_PAD, D), jnp.float32),
    )(agg, degi, dego, b, w)


def _tc_out(agg, degi, b):
    return pl.pallas_call(
        _tc_out_body,
        out_shape=jax.ShapeDtypeStruct((N_PAD, D), jnp.float32),
    )(agg, degi, b)


def kernel(x, edge_index, W1, b1, W2, b2, W3, b3):
    src = edge_index[0].astype(jnp.int32)
    dst = edge_index[1].astype(jnp.int32)
    pad = TRASH + jnp.arange(E_PAD - E, dtype=jnp.int32) % (N_PAD - N_NODES)
    src_p = jnp.concatenate([src, pad])
    dst_p = jnp.concatenate([dst, pad])
    srcm = src_p.reshape(-1, CH2)
    dstm = dst_p.reshape(-1, CH2)
    iota = jnp.arange(DEG_ROWS, dtype=jnp.int32).reshape(1, DEG_ROWS)
    zeros = jnp.zeros((128, 128), jnp.float32)

    dego_f, degi_f = _deg_kernel(src_p, dst_p, iota, zeros[:DEG_ROWS])
    dego = dego_f.reshape(NC, -1).sum(0).reshape(N_PAD, 1)
    degi = degi_f.reshape(NC, -1).sum(0).reshape(N_PAD, 1)

    x_pad = jnp.pad(x, ((0, N_PAD - N_NODES), (0, 0)))
    b1r, b2r, b3r = (b.reshape(1, D) for b in (b1, b2, b3))

    h = _tc_in(x_pad, dego, W1)
    agg = _edge_kernel(h, srcm, dstm, zeros)
    h = _tc_mid(agg, degi, dego, b1r, W2)
    agg = _edge_kernel(h, srcm, dstm, zeros)
    h = _tc_mid(agg, degi, dego, b2r, W3)
    agg = _edge_kernel(h, srcm, dstm, zeros)
    return _tc_out(agg, degi, b3r)[:N_NODES]


# speedup vs baseline: 1.2726x; 1.1741x over previous
import functools

import jax
import jax.numpy as jnp
from jax import lax
from jax.experimental import pallas as pl
from jax.experimental.pallas import tpu as pltpu
from jax.experimental.pallas import tpu_sc as plsc

N_NODES = 10000
D = 128
NC, NS = 2, 16
NW = NC * NS
N_PAD = 10240
TRASH = 10000
E = 320000
CH = 128
EROWS = 2560
CPT = EROWS // NW
E_PAD = EROWS * CH
RPT = N_PAD // NS
DEG_ROWS = N_PAD // 128
DEG_RPT = 8

_MESH = plsc.VectorSubcoreMesh(core_axis_name="c", subcore_axis_name="s")


E_PT = E_PAD // NW


def _deg_body(srcv, dstv, iota_hbm, zeros_hbm, out_o, out_i,
              src_v, dst_v, ho, hi, idx_v, tmp_v, sho, shi, sem1, sem2):
    c = lax.axis_index("c")
    s = lax.axis_index("s")
    wid = s * NC + c
    pltpu.sync_copy(zeros_hbm, ho)
    pltpu.sync_copy(zeros_hbm, hi)
    pltpu.sync_copy(zeros_hbm.at[pl.ds(0, DEG_RPT)], tmp_v)

    @pl.when(s < DEG_ROWS // DEG_RPT)
    def _zero_shared():
        pltpu.sync_copy(tmp_v, sho.at[pl.ds(s * DEG_RPT, DEG_RPT)])
        pltpu.sync_copy(tmp_v, shi.at[pl.ds(s * DEG_RPT, DEG_RPT)])

    pltpu.sync_copy(srcv.at[pl.ds(wid * E_PT, E_PT)], src_v)
    pltpu.sync_copy(dstv.at[pl.ds(wid * E_PT, E_PT)], dst_v)
    pltpu.sync_copy(iota_hbm, idx_v)
    ones = jnp.ones((16,), jnp.float32)

    def body(k, carry):
        s16 = src_v[pl.ds(k * 16, 16)]
        d16 = dst_v[pl.ds(k * 16, 16)]
        plsc.addupdate_scatter(ho, [s16 >> 7, s16 & 127], ones)
        plsc.addupdate_scatter(hi, [d16 >> 7, d16 & 127], ones)
        return carry

    lax.fori_loop(0, E_PT // 16, body, 0)
    plsc.subcore_barrier()
    pltpu.async_copy(ho, sho.at[idx_v.at[0]], sem1, add=True).wait()
    pltpu.async_copy(hi, shi.at[idx_v.at[0]], sem2, add=True).wait()
    plsc.subcore_barrier()

    @pl.when(s < DEG_ROWS // DEG_RPT)
    def _writeback():
        base = c * DEG_ROWS + s * DEG_RPT
        pltpu.sync_copy(sho.at[pl.ds(s * DEG_RPT, DEG_RPT)], tmp_v)
        pltpu.sync_copy(tmp_v, out_o.at[pl.ds(base, DEG_RPT)])
        pltpu.sync_copy(shi.at[pl.ds(s * DEG_RPT, DEG_RPT)], tmp_v)
        pltpu.sync_copy(tmp_v, out_i.at[pl.ds(base, DEG_RPT)])


_deg_kernel = functools.partial(
    pl.kernel,
    out_type=(jax.ShapeDtypeStruct((NC * DEG_ROWS, 128), jnp.float32),
              jax.ShapeDtypeStruct((NC * DEG_ROWS, 128), jnp.float32)),
    mesh=_MESH,
    scratch_types=[
        pltpu.VMEM((E_PT,), jnp.int32),
        pltpu.VMEM((E_PT,), jnp.int32),
        pltpu.VMEM((DEG_ROWS, 128), jnp.float32),
        pltpu.VMEM((DEG_ROWS, 128), jnp.float32),
        pltpu.VMEM((1, DEG_ROWS), jnp.int32),
        pltpu.VMEM((DEG_RPT, 128), jnp.float32),
        pltpu.VMEM_SHARED((DEG_ROWS, 128), jnp.float32),
        pltpu.VMEM_SHARED((DEG_ROWS, 128), jnp.float32),
        pltpu.SemaphoreType.DMA,
        pltpu.SemaphoreType.DMA,
    ],
    compiler_params=pltpu.CompilerParams(needs_layout_passes=False),
)(_deg_body)


CH2 = 128
CROWS_PT = E_PAD // CH2 // NW
BLK = 16


def _edge_body(h_hbm, srcm, dstm, zeros_hbm, out_hbm,
               src_v, dst_v, rows0_v, rows1_v, agg_sh,
               gsem0, gsem1, ssem0, ssem1, isem):
    c = lax.axis_index("c")
    s = lax.axis_index("s")
    wid = s * NC + c
    rows = (rows0_v, rows1_v)
    gsems = (gsem0, gsem1)
    ssems = (ssem0, ssem1)
    nblk = CROWS_PT // BLK
    pltpu.sync_copy(zeros_hbm, rows0_v)
    zds = [pltpu.async_copy(rows0_v, agg_sh.at[pl.ds(s * RPT + k * CH2, CH2)],
                            gsem0)
           for k in range(RPT // CH2)]
    i_st = (pltpu.async_copy(srcm.at[pl.ds(wid * CROWS_PT, BLK)],
                             src_v.at[pl.ds(0, BLK)], isem),
            pltpu.async_copy(dstm.at[pl.ds(wid * CROWS_PT, BLK)],
                             dst_v.at[pl.ds(0, BLK)], isem))
    for z in zds:
        z.wait()
    plsc.subcore_barrier()
    sd = [None, None]
    prev = None
    for t in range(CROWS_PT):
        blk, j = divmod(t, BLK)
        slot = (blk % 3) * BLK
        if j == 0:
            i_st[0].wait()
            i_st[1].wait()
            if blk + 1 < nblk:
                ns = ((blk + 1) % 3) * BLK
                nxt = wid * CROWS_PT + (blk + 1) * BLK
                i_st = (pltpu.async_copy(srcm.at[pl.ds(nxt, BLK)],
                                         src_v.at[pl.ds(ns, BLK)], isem),
                        pltpu.async_copy(dstm.at[pl.ds(nxt, BLK)],
                                         dst_v.at[pl.ds(ns, BLK)], isem))
        b = t & 1
        if sd[b] is not None:
            sd[b].wait()
        gd = pltpu.async_copy(h_hbm.at[src_v.at[slot + j]], rows[b], gsems[b])
        if prev is not None:
            pgd, pb, prow = prev
            pgd.wait()
            sd[pb] = pltpu.async_copy(rows[pb], agg_sh.at[dst_v.at[prow]],
                                      ssems[pb], add=True)
        prev = (gd, b, slot + j)
    pgd, pb, prow = prev
    pgd.wait()
    sd[pb] = pltpu.async_copy(rows[pb], agg_sh.at[dst_v.at[prow]],
                              ssems[pb], add=True)
    sd[0].wait()
    sd[1].wait()
    plsc.subcore_barrier()
    base = c * N_PAD + s * RPT
    wd = [None, None]
    for k in range(RPT // CH2):
        b = k & 1
        if wd[b] is not None:
            wd[b].wait()
        pltpu.sync_copy(agg_sh.at[pl.ds(s * RPT + k * CH2, CH2)], rows[b])
        wd[b] = pltpu.async_copy(rows[b],
                                 out_hbm.at[pl.ds(base + k * CH2, CH2)],
                                 gsems[b])
    wd[0].wait()
    wd[1].wait()


_edge_kernel = functools.partial(
    pl.kernel,
    out_type=jax.ShapeDtypeStruct((NC * N_PAD, D), jnp.float32),
    mesh=_MESH,
    scratch_types=[
        pltpu.VMEM((3 * BLK, CH2), jnp.int32),
        pltpu.VMEM((3 * BLK, CH2), jnp.int32),
        pltpu.VMEM((CH2, D), jnp.float32),
        pltpu.VMEM((CH2, D), jnp.float32),
        pltpu.VMEM_SHARED((N_PAD, D), jnp.float32),
        pltpu.SemaphoreType.DMA,
        pltpu.SemaphoreType.DMA,
        pltpu.SemaphoreType.DMA,
        pltpu.SemaphoreType.DMA,
        pltpu.SemaphoreType.DMA,
    ],
)(_edge_body)


def _norm(deg):
    return jnp.where(deg > 0, lax.rsqrt(jnp.maximum(deg, 1.0)), 0.0)


def _sum_parts(agg_ref):
    return agg_ref[0:N_PAD, :] + agg_ref[N_PAD:, :]


def _tc_in_body(x_ref, dego_ref, w_ref, o_ref):
    h = x_ref[...] * _norm(dego_ref[...])
    o_ref[...] = jnp.dot(h, w_ref[...], preferred_element_type=jnp.float32)


def _tc_mid_body(agg_ref, degi_ref, dego_ref, b_ref, w_ref, o_ref):
    agg = _sum_parts(agg_ref)
    h = jnp.maximum(agg * _norm(degi_ref[...]) + b_ref[...], 0.0)
    h = h * _norm(dego_ref[...])
    o_ref[...] = jnp.dot(h, w_ref[...], preferred_element_type=jnp.float32)


def _tc_out_body(agg_ref, degi_ref, b_ref, o_ref):
    o_ref[...] = _sum_parts(agg_ref) * _norm(degi_ref[...]) + b_ref[...]


def _tc_in(x, dego, w):
    return pl.pallas_call(
        _tc_in_body,
        out_shape=jax.ShapeDtypeStruct((N_PAD, D), jnp.float32),
    )(x, dego, w)


def _tc_mid(agg, degi, dego, b, w):
    return pl.pallas_call(
        _tc_mid_body,
        out_shape=jax.ShapeDtypeStruct((N_PAD, D), jnp.float32),
    )(agg, degi, dego, b, w)


def _tc_out(agg, degi, b):
    return pl.pallas_call(
        _tc_out_body,
        out_shape=jax.ShapeDtypeStruct((N_PAD, D), jnp.float32),
    )(agg, degi, b)


def kernel(x, edge_index, W1, b1, W2, b2, W3, b3):
    src = edge_index[0].astype(jnp.int32)
    dst = edge_index[1].astype(jnp.int32)
    pad = TRASH + jnp.arange(E_PAD - E, dtype=jnp.int32) % (N_PAD - N_NODES)
    src_p = jnp.concatenate([src, pad])
    dst_p = jnp.concatenate([dst, pad])
    srcm = src_p.reshape(-1, CH2)
    dstm = dst_p.reshape(-1, CH2)
    iota = jnp.arange(DEG_ROWS, dtype=jnp.int32).reshape(1, DEG_ROWS)
    zeros = jnp.zeros((128, 128), jnp.float32)

    dego_f, degi_f = _deg_kernel(src_p, dst_p, iota, zeros[:DEG_ROWS])
    dego = dego_f.reshape(NC, -1).sum(0).reshape(N_PAD, 1)
    degi = degi_f.reshape(NC, -1).sum(0).reshape(N_PAD, 1)

    x_pad = jnp.pad(x, ((0, N_PAD - N_NODES), (0, 0)))
    b1r, b2r, b3r = (b.reshape(1, D) for b in (b1, b2, b3))

    h = _tc_in(x_pad, dego, W1)
    agg = _edge_kernel(h, srcm, dstm, zeros)
    h = _tc_mid(agg, degi, dego, b1r, W2)
    agg = _edge_kernel(h, srcm, dstm, zeros)
    h = _tc_mid(agg, degi, dego, b2r, W3)
    agg = _edge_kernel(h, srcm, dstm, zeros)
    return _tc_out(agg, degi, b3r)[:N_NODES]
